# baseline (speedup 1.0000x reference)
import functools

import jax
import jax.numpy as jnp
from jax import lax
from jax.experimental import pallas as pl
from jax.experimental.pallas import tpu as pltpu
from jax.experimental.pallas import tpu_sc as plsc

DIM = 32
NNB = 16
NRELP = 48
NC, NS, L = 2, 16, 16
NW = NC * NS


def _mesh():
    return plsc.VectorSubcoreMesh(core_axis_name="c", subcore_axis_name="s")


def _gather_aggregate(u, v, adj, rel, usr_table, ent_table, rtT):
    B = v.shape[0]
    bpw = B // NW

    @functools.partial(
        pl.kernel,
        out_type=(
            jax.ShapeDtypeStruct((B, DIM), jnp.float32),
            jax.ShapeDtypeStruct((B, DIM), jnp.float32),
            jax.ShapeDtypeStruct((B, NNB * DIM), jnp.float32),
            jax.ShapeDtypeStruct((B, NNB), jnp.float32),
        ),
        mesh=_mesh(),
        scratch_types=[
            pltpu.VMEM((bpw,), jnp.int32),
            pltpu.VMEM((bpw, DIM), jnp.float32),
            pltpu.VMEM((DIM, NRELP), jnp.float32),
            pltpu.VMEM((NRELP,), jnp.float32),
            pltpu.VMEM((bpw,), jnp.int32),
            pltpu.VMEM((bpw, NNB), jnp.int32),
            pltpu.VMEM((bpw * NNB,), jnp.int32),
            pltpu.VMEM((bpw, NNB), jnp.int32),
            pltpu.VMEM((bpw, DIM), jnp.float32),
            pltpu.VMEM((bpw * NNB, NNB), jnp.int32),
            pltpu.VMEM((bpw * NNB, NNB), jnp.int32),
            pltpu.VMEM((4, NNB * NNB), jnp.int32),
            pltpu.VMEM((4, NNB * NNB, DIM), jnp.float32),
            pltpu.VMEM((4, NNB, DIM), jnp.float32),
            pltpu.VMEM((4, NNB * DIM), jnp.float32),
            pltpu.VMEM((bpw, DIM), jnp.float32),
            pltpu.VMEM((bpw, NNB), jnp.float32),
            pltpu.VMEM((L,), jnp.float32),
        ] + [pltpu.SemaphoreType.DMA] * 12,
        compiler_params=pltpu.CompilerParams(
            use_tc_tiling_on_sc=False, needs_layout_passes=False),
    )
    def k(u_hbm, v_hbm, adj_hbm, rel_hbm, usr_hbm, ent_hbm, rtT_hbm,
          ue_hbm, sum0_hbm, sum1_hbm, w0_hbm,
          UL, UE, RT, ECb, VL, E1, E1F, Q0, SV0, E2F, Q1F, XIF, Xb, SV1b,
          SUM1b, SUM0, W0B, wbuf, *sems):
        semX = sems[0:4]
        semS = sems[4:8]
        semW = sems[8:12]
        wid = lax.axis_index("s") * NC + lax.axis_index("c")
        base = wid * bpw

        pltpu.sync_copy(u_hbm.at[pl.ds(base, bpw)], UL)
        pltpu.sync_copy(v_hbm.at[pl.ds(base, bpw)], VL)
        pltpu.sync_copy(rtT_hbm, RT)
        pltpu.sync_copy(usr_hbm.at[UL], UE)
        pltpu.sync_copy(adj_hbm.at[VL], E1)
        pltpu.sync_copy(rel_hbm.at[VL], Q0)
        pltpu.sync_copy(ent_hbm.at[VL], SV0)
        pltpu.sync_copy(UE, ue_hbm.at[pl.ds(base, bpw)])

        def flatten(i, carry):
            E1F[pl.ds(i * NNB, NNB)] = E1[i, :]
            return carry
        lax.fori_loop(0, bpw, flatten, 0)

        pltpu.sync_copy(adj_hbm.at[E1F], E2F)
        pltpu.sync_copy(rel_hbm.at[E1F], Q1F)

        def exp_scores(b_vec):
            u0 = jnp.zeros((L,), jnp.float32)
            u1 = jnp.zeros((L,), jnp.float32)
            u2 = jnp.zeros((L,), jnp.float32)
            for d in range(DIM):
                ud = plsc.load_gather(
                    UE, [b_vec, jnp.zeros((L,), jnp.int32) + d])
                u0 = u0 + ud * RT[d, 0:L]
                u1 = u1 + ud * RT[d, L:2 * L]
                u2 = u2 + ud * RT[d, 2 * L:3 * L]
            ECb[pl.ds(0, L)] = jnp.exp(u0)
            ECb[pl.ds(L, L)] = jnp.exp(u1)
            ECb[pl.ds(2 * L, L)] = jnp.exp(u2)

        def seg_weights(q):
            e = plsc.load_gather(ECb, [q])
            s = jnp.sum(e)
            wbuf[...] = e
            return (jnp.zeros((L,), jnp.float32) + 1.0) / (
                jnp.zeros((L,), jnp.float32) + s)

        def fire(b, j):
            for p in range(NNB):
                XIF[j, pl.ds(p * NNB, NNB)] = E2F[b * NNB + p, :]
            pltpu.async_copy(ent_hbm.at[XIF.at[j]], Xb.at[j], semX[j])
            pltpu.async_copy(ent_hbm.at[E1F.at[pl.ds(b * NNB, NNB)]],
                             SV1b.at[j], semS[j])

        fire(0, 0)
        fire(1, 1)
        fire(2, 2)
        fire(3, 3)

        def outer(i, carry):
            for j in range(4):
                b = i * 4 + j
                b_vec = jnp.zeros((L,), jnp.int32) + b
                exp_scores(b_vec)
                pltpu.make_async_copy(
                    ent_hbm.at[XIF.at[j]], Xb.at[j], semX[j]).wait()
                pltpu.make_async_copy(
                    ent_hbm.at[E1F.at[pl.ds(b * NNB, NNB)]],
                    SV1b.at[j], semS[j]).wait()

                @pl.when(b >= 4)
                def _():
                    pltpu.make_async_copy(
                        SUM1b.at[j], sum1_hbm.at[base + b - 4], semW[j]).wait()

                for p in range(NNB):
                    rs = seg_weights(Q1F[b * NNB + p, :])
                    acc0 = jnp.zeros((L,), jnp.float32)
                    acc1 = jnp.zeros((L,), jnp.float32)
                    for kk in range(NNB):
                        bk = plsc.load_gather(
                            wbuf, [jnp.zeros((L,), jnp.int32) + kk])
                        acc0 = acc0 + bk * Xb[j, p * NNB + kk, 0:L]
                        acc1 = acc1 + bk * Xb[j, p * NNB + kk, L:DIM]
                    SUM1b[j, pl.ds(p * DIM, L)] = acc0 * rs + SV1b[j, p, 0:L]
                    SUM1b[j, pl.ds(p * DIM + L, L)] = (
                        acc1 * rs + SV1b[j, p, L:DIM])
                pltpu.async_copy(SUM1b.at[j], sum1_hbm.at[base + b], semW[j])
                rs0 = seg_weights(Q0[b, :])
                a0 = jnp.zeros((L,), jnp.float32)
                a1 = jnp.zeros((L,), jnp.float32)
                for kk in range(NNB):
                    bk = plsc.load_gather(
                        wbuf, [jnp.zeros((L,), jnp.int32) + kk])
                    a0 = a0 + bk * SV1b[j, kk, 0:L]
                    a1 = a1 + bk * SV1b[j, kk, L:DIM]
                W0B[b, :] = wbuf[...] * rs0
                SUM0[b, 0:L] = a0 * rs0 + SV0[b, 0:L]
                SUM0[b, L:DIM] = a1 * rs0 + SV0[b, L:DIM]

                @pl.when(b + 4 < bpw)
                def _():
                    fire(b + 4, j)
            return carry

        lax.fori_loop(0, bpw // 4, outer, 0)
        for j in range(4):
            pltpu.make_async_copy(
                SUM1b.at[j], sum1_hbm.at[base + bpw - 4 + j], semW[j]).wait()
        pltpu.sync_copy(SUM0, sum0_hbm.at[pl.ds(base, bpw)])
        pltpu.sync_copy(W0B, w0_hbm.at[pl.ds(base, bpw)])

    return k(u, v, adj, rel, usr_table, ent_table, rtT)


def _dense_finish(user_emb, sum0, sum1_2d, w0, W0T, b0, W1T, b1):
    B = user_emb.shape[0]
    BB = 512
    grid = B // BB

    def body(ue_ref, s0_ref, s1_ref, w0_ref, w0t_ref, b0_ref, w1t_ref, b1_ref,
             out_ref):
        w0t = w0t_ref[...]
        b0v = b0_ref[...]
        w0w = w0_ref[...]
        aggtop = jnp.zeros((BB, DIM), jnp.float32)
        for kk in range(NNB):
            h1k = jax.nn.sigmoid(
                jnp.dot(s1_ref[:, kk * DIM:(kk + 1) * DIM], w0t,
                        preferred_element_type=jnp.float32) + b0v
            )
            aggtop = aggtop + w0w[:, kk:kk + 1] * h1k
        h0 = jax.nn.sigmoid(
            jnp.dot(s0_ref[...], w0t, preferred_element_type=jnp.float32) + b0v
        )
        item = jnp.tanh(
            jnp.dot(h0 + aggtop, w1t_ref[...], preferred_element_type=jnp.float32)
            + b1_ref[...]
        )
        out_ref[...] = jax.nn.sigmoid(jnp.sum(ue_ref[...] * item, axis=1))

    return pl.pallas_call(
        body,
        grid=(grid,),
        in_specs=[
            pl.BlockSpec((BB, DIM), lambda i: (i, 0)),
            pl.BlockSpec((BB, DIM), lambda i: (i, 0)),
            pl.BlockSpec((BB, NNB * DIM), lambda i: (i, 0)),
            pl.BlockSpec((BB, NNB), lambda i: (i, 0)),
            pl.BlockSpec((DIM, DIM), lambda i: (0, 0)),
            pl.BlockSpec((1, DIM), lambda i: (0, 0)),
            pl.BlockSpec((DIM, DIM), lambda i: (0, 0)),
            pl.BlockSpec((1, DIM), lambda i: (0, 0)),
        ],
        out_specs=pl.BlockSpec((BB,), lambda i: (i,)),
        out_shape=jax.ShapeDtypeStruct((B,), jnp.float32),
    )(user_emb, sum0, sum1_2d, w0, W0T, b0, W1T, b1)


def kernel(u, v, adj, rel, train_mode, usr_table, ent_table, rel_table,
           agg_W0, agg_b0, agg_W1, agg_b1):
    del train_mode
    u = u.astype(jnp.int32)
    v = v.astype(jnp.int32)
    adj = adj.astype(jnp.int32)
    rel = rel.astype(jnp.int32)

    rtT = jnp.zeros((DIM, NRELP), jnp.float32).at[:, :rel_table.shape[0]].set(
        rel_table.T)

    user_emb, sum0, sum1, w0 = _gather_aggregate(
        u, v, adj, rel, usr_table, ent_table, rtT)

    return _dense_finish(
        user_emb, sum0, sum1, w0,
        agg_W0.T, agg_b0.reshape(1, DIM), agg_W1.T, agg_b1.reshape(1, DIM))


# speedup vs baseline: 1.0992x; 1.0992x over previous
import functools

import jax
import jax.numpy as jnp
from jax import lax
from jax.experimental import pallas as pl
from jax.experimental.pallas import tpu as pltpu
from jax.experimental.pallas import tpu_sc as plsc

DIM = 32
NNB = 16
NRELP = 48
NC, NS, L = 2, 16, 16
NW = NC * NS


def _mesh():
    return plsc.VectorSubcoreMesh(core_axis_name="c", subcore_axis_name="s")


def _gather_aggregate(u, v, adj, rel, usr_table, ent_table, rtT):
    B = v.shape[0]
    bpw = B // NW

    @functools.partial(
        pl.kernel,
        out_type=(
            jax.ShapeDtypeStruct((B, DIM), jnp.float32),
            jax.ShapeDtypeStruct((B, DIM), jnp.float32),
            jax.ShapeDtypeStruct((B, NNB * DIM), jnp.float32),
            jax.ShapeDtypeStruct((B, NNB), jnp.float32),
        ),
        mesh=_mesh(),
        scratch_types=[
            pltpu.VMEM((bpw,), jnp.int32),
            pltpu.VMEM((bpw, DIM), jnp.float32),
            pltpu.VMEM((DIM, NRELP), jnp.float32),
            pltpu.VMEM((NRELP,), jnp.float32),
            pltpu.VMEM((bpw,), jnp.int32),
            pltpu.VMEM((bpw, NNB), jnp.int32),
            pltpu.VMEM((bpw * NNB,), jnp.int32),
            pltpu.VMEM((bpw, NNB), jnp.int32),
            pltpu.VMEM((bpw, DIM), jnp.float32),
            pltpu.VMEM((bpw * NNB, NNB), jnp.int32),
            pltpu.VMEM((bpw * NNB, NNB), jnp.int32),
            pltpu.VMEM((2, NNB * NNB), jnp.int32),
            pltpu.VMEM((2, NNB * NNB, DIM), jnp.float32),
            pltpu.VMEM((2, NNB, DIM), jnp.float32),
            pltpu.VMEM((2, NNB * DIM), jnp.float32),
            pltpu.VMEM((bpw, DIM), jnp.float32),
            pltpu.VMEM((bpw, NNB), jnp.float32),
            pltpu.VMEM((L,), jnp.float32),
        ] + [pltpu.SemaphoreType.DMA] * 6,
        compiler_params=pltpu.CompilerParams(
            use_tc_tiling_on_sc=False, needs_layout_passes=False),
    )
    def k(u_hbm, v_hbm, adj_hbm, rel_hbm, usr_hbm, ent_hbm, rtT_hbm,
          ue_hbm, sum0_hbm, sum1_hbm, w0_hbm,
          UL, UE, RT, ECb, VL, E1, E1F, Q0, SV0, E2F, Q1F, XIF, Xb, SV1b,
          SUM1b, SUM0, W0B, wbuf, *sems):
        semX = sems[0:2]
        semS = sems[2:4]
        semW = sems[4:6]
        wid = lax.axis_index("s") * NC + lax.axis_index("c")
        base = wid * bpw

        pltpu.sync_copy(u_hbm.at[pl.ds(base, bpw)], UL)
        pltpu.sync_copy(v_hbm.at[pl.ds(base, bpw)], VL)
        pltpu.sync_copy(rtT_hbm, RT)
        pltpu.sync_copy(usr_hbm.at[UL], UE)
        pltpu.sync_copy(adj_hbm.at[VL], E1)
        pltpu.sync_copy(rel_hbm.at[VL], Q0)
        pltpu.sync_copy(ent_hbm.at[VL], SV0)
        pltpu.sync_copy(UE, ue_hbm.at[pl.ds(base, bpw)])

        def flatten(i, carry):
            E1F[pl.ds(i * NNB, NNB)] = E1[i, :]
            return carry
        lax.fori_loop(0, bpw, flatten, 0)

        pltpu.sync_copy(adj_hbm.at[E1F], E2F)
        pltpu.sync_copy(rel_hbm.at[E1F], Q1F)

        def exp_scores(b_vec):
            u0 = jnp.zeros((L,), jnp.float32)
            u1 = jnp.zeros((L,), jnp.float32)
            u2 = jnp.zeros((L,), jnp.float32)
            for d in range(DIM):
                ud = plsc.load_gather(
                    UE, [b_vec, jnp.zeros((L,), jnp.int32) + d])
                u0 = u0 + ud * RT[d, 0:L]
                u1 = u1 + ud * RT[d, L:2 * L]
                u2 = u2 + ud * RT[d, 2 * L:3 * L]
            ECb[pl.ds(0, L)] = jnp.exp(u0)
            ECb[pl.ds(L, L)] = jnp.exp(u1)
            ECb[pl.ds(2 * L, L)] = jnp.exp(u2)

        def seg_weights(q):
            e = plsc.load_gather(ECb, [q])
            s = jnp.sum(e)
            wbuf[...] = e
            return (jnp.zeros((L,), jnp.float32) + 1.0) / (
                jnp.zeros((L,), jnp.float32) + s)

        def fire(b, j):
            for p in range(NNB):
                XIF[j, pl.ds(p * NNB, NNB)] = E2F[b * NNB + p, :]
            pltpu.async_copy(ent_hbm.at[XIF.at[j]], Xb.at[j], semX[j])
            pltpu.async_copy(ent_hbm.at[E1F.at[pl.ds(b * NNB, NNB)]],
                             SV1b.at[j], semS[j])

        fire(0, 0)
        fire(1, 1)

        def outer(i, carry):
            for j in range(2):
                b = i * 2 + j
                b_vec = jnp.zeros((L,), jnp.int32) + b
                exp_scores(b_vec)
                pltpu.make_async_copy(
                    ent_hbm.at[XIF.at[j]], Xb.at[j], semX[j]).wait()
                pltpu.make_async_copy(
                    ent_hbm.at[E1F.at[pl.ds(b * NNB, NNB)]],
                    SV1b.at[j], semS[j]).wait()

                @pl.when(b >= 2)
                def _():
                    pltpu.make_async_copy(
                        SUM1b.at[j], sum1_hbm.at[base + b - 2], semW[j]).wait()

                for p in range(NNB):
                    rs = seg_weights(Q1F[b * NNB + p, :])
                    acc0 = jnp.zeros((L,), jnp.float32)
                    acc1 = jnp.zeros((L,), jnp.float32)
                    for kk in range(NNB):
                        bk = plsc.load_gather(
                            wbuf, [jnp.zeros((L,), jnp.int32) + kk])
                        acc0 = acc0 + bk * Xb[j, p * NNB + kk, 0:L]
                        acc1 = acc1 + bk * Xb[j, p * NNB + kk, L:DIM]
                    SUM1b[j, pl.ds(p * DIM, L)] = acc0 * rs + SV1b[j, p, 0:L]
                    SUM1b[j, pl.ds(p * DIM + L, L)] = (
                        acc1 * rs + SV1b[j, p, L:DIM])
                pltpu.async_copy(SUM1b.at[j], sum1_hbm.at[base + b], semW[j])
                rs0 = seg_weights(Q0[b, :])
                a0 = jnp.zeros((L,), jnp.float32)
                a1 = jnp.zeros((L,), jnp.float32)
                for kk in range(NNB):
                    bk = plsc.load_gather(
                        wbuf, [jnp.zeros((L,), jnp.int32) + kk])
                    a0 = a0 + bk * SV1b[j, kk, 0:L]
                    a1 = a1 + bk * SV1b[j, kk, L:DIM]
                W0B[b, :] = wbuf[...] * rs0
                SUM0[b, 0:L] = a0 * rs0 + SV0[b, 0:L]
                SUM0[b, L:DIM] = a1 * rs0 + SV0[b, L:DIM]

                @pl.when(b + 2 < bpw)
                def _():
                    fire(b + 2, j)
            return carry

        lax.fori_loop(0, bpw // 2, outer, 0)
        for j in range(2):
            pltpu.make_async_copy(
                SUM1b.at[j], sum1_hbm.at[base + bpw - 2 + j], semW[j]).wait()
        pltpu.sync_copy(SUM0, sum0_hbm.at[pl.ds(base, bpw)])
        pltpu.sync_copy(W0B, w0_hbm.at[pl.ds(base, bpw)])

    return k(u, v, adj, rel, usr_table, ent_table, rtT)


def _dense_finish(user_emb, sum0, sum1_2d, w0, W0T, b0, W1T, b1):
    B = user_emb.shape[0]
    BB = 512
    grid = B // BB

    def body(ue_ref, s0_ref, s1_ref, w0_ref, w0t_ref, b0_ref, w1t_ref, b1_ref,
             out_ref):
        w0t = w0t_ref[...]
        b0v = b0_ref[...]
        w0w = w0_ref[...]
        s1 = s1_ref[...].reshape(BB, NNB * DIM)
        aggtop = jnp.zeros((BB, DIM), jnp.float32)
        for kk in range(NNB):
            h1k = jax.nn.sigmoid(
                jnp.dot(s1[:, kk * DIM:(kk + 1) * DIM], w0t,
                        preferred_element_type=jnp.float32) + b0v
            )
            aggtop = aggtop + w0w[:, kk:kk + 1] * h1k
        h0 = jax.nn.sigmoid(
            jnp.dot(s0_ref[...], w0t, preferred_element_type=jnp.float32) + b0v
        )
        item = jnp.tanh(
            jnp.dot(h0 + aggtop, w1t_ref[...], preferred_element_type=jnp.float32)
            + b1_ref[...]
        )
        out_ref[...] = jax.nn.sigmoid(jnp.sum(ue_ref[...] * item, axis=1))

    return pl.pallas_call(
        body,
        grid=(grid,),
        in_specs=[
            pl.BlockSpec((BB, DIM), lambda i: (i, 0)),
            pl.BlockSpec((BB, DIM), lambda i: (i, 0)),
            pl.BlockSpec((BB * NNB * DIM,), lambda i: (i,)),
            pl.BlockSpec((BB, NNB), lambda i: (i, 0)),
            pl.BlockSpec((DIM, DIM), lambda i: (0, 0)),
            pl.BlockSpec((1, DIM), lambda i: (0, 0)),
            pl.BlockSpec((DIM, DIM), lambda i: (0, 0)),
            pl.BlockSpec((1, DIM), lambda i: (0, 0)),
        ],
        out_specs=pl.BlockSpec((BB,), lambda i: (i,)),
        out_shape=jax.ShapeDtypeStruct((B,), jnp.float32),
    )(user_emb, sum0, sum1_2d, w0, W0T, b0, W1T, b1)


def kernel(u, v, adj, rel, train_mode, usr_table, ent_table, rel_table,
           agg_W0, agg_b0, agg_W1, agg_b1):
    del train_mode
    u = u.astype(jnp.int32)
    v = v.astype(jnp.int32)
    adj = adj.astype(jnp.int32)
    rel = rel.astype(jnp.int32)

    rtT = jnp.zeros((DIM, NRELP), jnp.float32).at[:, :rel_table.shape[0]].set(
        rel_table.T)

    user_emb, sum0, sum1, w0 = _gather_aggregate(
        u, v, adj, rel, usr_table, ent_table, rtT)

    return _dense_finish(
        user_emb, sum0, sum1.reshape(-1), w0,
        agg_W0.T, agg_b0.reshape(1, DIM), agg_W1.T, agg_b1.reshape(1, DIM))


# speedup vs baseline: 1.1352x; 1.0328x over previous
import functools

import jax
import jax.numpy as jnp
from jax import lax
from jax.experimental import pallas as pl
from jax.experimental.pallas import tpu as pltpu
from jax.experimental.pallas import tpu_sc as plsc

DIM = 32
NNB = 16
NRELP = 48
NC, NS, L = 2, 16, 16
NW = NC * NS


def _mesh():
    return plsc.VectorSubcoreMesh(core_axis_name="c", subcore_axis_name="s")


def _gather_aggregate(u, v, adj, rel, usr_table, ent_table, rtT):
    B = v.shape[0]
    bpw = B // NW

    @functools.partial(
        pl.kernel,
        out_type=(
            jax.ShapeDtypeStruct((B, DIM), jnp.float32),
            jax.ShapeDtypeStruct((B, DIM), jnp.float32),
            jax.ShapeDtypeStruct((B, NNB * DIM), jnp.float32),
            jax.ShapeDtypeStruct((B, NNB), jnp.float32),
        ),
        mesh=_mesh(),
        scratch_types=[
            pltpu.VMEM((bpw,), jnp.int32),
            pltpu.VMEM((bpw, DIM), jnp.float32),
            pltpu.VMEM((DIM, NRELP), jnp.float32),
            pltpu.VMEM((NRELP,), jnp.float32),
            pltpu.VMEM((bpw,), jnp.int32),
            pltpu.VMEM((bpw, NNB), jnp.int32),
            pltpu.VMEM((bpw * NNB,), jnp.int32),
            pltpu.VMEM((bpw, NNB), jnp.int32),
            pltpu.VMEM((bpw, DIM), jnp.bfloat16),
            pltpu.VMEM((bpw * NNB, NNB), jnp.int32),
            pltpu.VMEM((bpw * NNB, NNB), jnp.int32),
            pltpu.VMEM((2, NNB * NNB), jnp.int32),
            pltpu.VMEM((2, NNB * NNB, DIM), jnp.bfloat16),
            pltpu.VMEM((2, NNB, DIM), jnp.bfloat16),
            pltpu.VMEM((2, NNB * DIM), jnp.float32),
            pltpu.VMEM((bpw, DIM), jnp.float32),
            pltpu.VMEM((bpw, NNB), jnp.float32),
            pltpu.VMEM((L,), jnp.float32),
        ] + [pltpu.SemaphoreType.DMA] * 6,
        compiler_params=pltpu.CompilerParams(
            use_tc_tiling_on_sc=False, needs_layout_passes=False),
    )
    def k(u_hbm, v_hbm, adj_hbm, rel_hbm, usr_hbm, ent_hbm, rtT_hbm,
          ue_hbm, sum0_hbm, sum1_hbm, w0_hbm,
          UL, UE, RT, ECb, VL, E1, E1F, Q0, SV0, E2F, Q1F, XIF, Xb, SV1b,
          SUM1b, SUM0, W0B, wbuf, *sems):
        semX = sems[0:2]
        semS = sems[2:4]
        semW = sems[4:6]
        wid = lax.axis_index("s") * NC + lax.axis_index("c")
        base = wid * bpw

        pltpu.sync_copy(u_hbm.at[pl.ds(base, bpw)], UL)
        pltpu.sync_copy(v_hbm.at[pl.ds(base, bpw)], VL)
        pltpu.sync_copy(rtT_hbm, RT)
        pltpu.sync_copy(usr_hbm.at[UL], UE)
        pltpu.sync_copy(adj_hbm.at[VL], E1)
        pltpu.sync_copy(rel_hbm.at[VL], Q0)
        pltpu.sync_copy(ent_hbm.at[VL], SV0)
        pltpu.sync_copy(UE, ue_hbm.at[pl.ds(base, bpw)])

        def flatten(i, carry):
            E1F[pl.ds(i * NNB, NNB)] = E1[i, :]
            return carry
        lax.fori_loop(0, bpw, flatten, 0)

        pltpu.sync_copy(adj_hbm.at[E1F], E2F)
        pltpu.sync_copy(rel_hbm.at[E1F], Q1F)

        def exp_scores(b_vec):
            u0 = jnp.zeros((L,), jnp.float32)
            u1 = jnp.zeros((L,), jnp.float32)
            u2 = jnp.zeros((L,), jnp.float32)
            for d in range(DIM):
                ud = plsc.load_gather(
                    UE, [b_vec, jnp.zeros((L,), jnp.int32) + d])
                u0 = u0 + ud * RT[d, 0:L]
                u1 = u1 + ud * RT[d, L:2 * L]
                u2 = u2 + ud * RT[d, 2 * L:3 * L]
            ECb[pl.ds(0, L)] = jnp.exp(u0)
            ECb[pl.ds(L, L)] = jnp.exp(u1)
            ECb[pl.ds(2 * L, L)] = jnp.exp(u2)

        def seg_weights(q):
            e = plsc.load_gather(ECb, [q])
            s = jnp.sum(e)
            wbuf[...] = e
            return (jnp.zeros((L,), jnp.float32) + 1.0) / (
                jnp.zeros((L,), jnp.float32) + s)

        def fire(b, j):
            for p in range(NNB):
                XIF[j, pl.ds(p * NNB, NNB)] = E2F[b * NNB + p, :]
            pltpu.async_copy(ent_hbm.at[XIF.at[j]], Xb.at[j], semX[j])
            pltpu.async_copy(ent_hbm.at[E1F.at[pl.ds(b * NNB, NNB)]],
                             SV1b.at[j], semS[j])

        fire(0, 0)
        fire(1, 1)

        def outer(i, carry):
            for j in range(2):
                b = i * 2 + j
                b_vec = jnp.zeros((L,), jnp.int32) + b
                exp_scores(b_vec)
                pltpu.make_async_copy(
                    ent_hbm.at[XIF.at[j]], Xb.at[j], semX[j]).wait()
                pltpu.make_async_copy(
                    ent_hbm.at[E1F.at[pl.ds(b * NNB, NNB)]],
                    SV1b.at[j], semS[j]).wait()

                @pl.when(b >= 2)
                def _():
                    pltpu.make_async_copy(
                        SUM1b.at[j], sum1_hbm.at[base + b - 2], semW[j]).wait()

                for p in range(NNB):
                    rs = seg_weights(Q1F[b * NNB + p, :])
                    acc0 = jnp.zeros((L,), jnp.float32)
                    acc1 = jnp.zeros((L,), jnp.float32)
                    for kk in range(NNB):
                        bk = plsc.load_gather(
                            wbuf, [jnp.zeros((L,), jnp.int32) + kk])
                        xa, xo = plsc.unpack(
                            Xb[j, p * NNB + kk, 0:DIM],
                            format=plsc.PackFormat.INTERLEAVED)
                        acc0 = acc0 + bk * xa
                        acc1 = acc1 + bk * xo
                    sva, svo = plsc.unpack(
                        SV1b[j, p, 0:DIM], format=plsc.PackFormat.INTERLEAVED)
                    SUM1b[j, pl.ds(p * DIM, L)] = acc0 * rs + sva
                    SUM1b[j, pl.ds(p * DIM + L, L)] = acc1 * rs + svo
                pltpu.async_copy(SUM1b.at[j], sum1_hbm.at[base + b], semW[j])
                rs0 = seg_weights(Q0[b, :])
                a0 = jnp.zeros((L,), jnp.float32)
                a1 = jnp.zeros((L,), jnp.float32)
                for kk in range(NNB):
                    bk = plsc.load_gather(
                        wbuf, [jnp.zeros((L,), jnp.int32) + kk])
                    sa, so = plsc.unpack(
                        SV1b[j, kk, 0:DIM], format=plsc.PackFormat.INTERLEAVED)
                    a0 = a0 + bk * sa
                    a1 = a1 + bk * so
                W0B[b, :] = wbuf[...] * rs0
                s0a, s0o = plsc.unpack(
                    SV0[b, 0:DIM], format=plsc.PackFormat.INTERLEAVED)
                SUM0[b, 0:L] = a0 * rs0 + s0a
                SUM0[b, L:DIM] = a1 * rs0 + s0o

                @pl.when(b + 2 < bpw)
                def _():
                    fire(b + 2, j)
            return carry

        lax.fori_loop(0, bpw // 2, outer, 0)
        for j in range(2):
            pltpu.make_async_copy(
                SUM1b.at[j], sum1_hbm.at[base + bpw - 2 + j], semW[j]).wait()
        pltpu.sync_copy(SUM0, sum0_hbm.at[pl.ds(base, bpw)])
        pltpu.sync_copy(W0B, w0_hbm.at[pl.ds(base, bpw)])

    return k(u, v, adj, rel, usr_table, ent_table, rtT)


def _dense_finish(user_emb, sum0, sum1_2d, w0, W0T, b0, W1T, b1):
    B = user_emb.shape[0]
    BB = 512
    grid = B // BB

    def body(ue_ref, s0_ref, s1_ref, w0_ref, w0t_ref, b0_ref, w1t_ref, b1_ref,
             out_ref):
        w0t = w0t_ref[...]
        b0v = b0_ref[...]
        w0w = w0_ref[...]
        s1 = s1_ref[...].reshape(BB, NNB * DIM)
        aggtop = jnp.zeros((BB, DIM), jnp.float32)
        for kk in range(NNB):
            h1k = jax.nn.sigmoid(
                jnp.dot(s1[:, kk * DIM:(kk + 1) * DIM], w0t,
                        preferred_element_type=jnp.float32) + b0v
            )
            aggtop = aggtop + w0w[:, kk:kk + 1] * h1k
        h0 = jax.nn.sigmoid(
            jnp.dot(s0_ref[...], w0t, preferred_element_type=jnp.float32) + b0v
        )
        item = jnp.tanh(
            jnp.dot(h0 + aggtop, w1t_ref[...], preferred_element_type=jnp.float32)
            + b1_ref[...]
        )
        out_ref[...] = jax.nn.sigmoid(jnp.sum(ue_ref[...] * item, axis=1))

    return pl.pallas_call(
        body,
        grid=(grid,),
        in_specs=[
            pl.BlockSpec((BB, DIM), lambda i: (i, 0)),
            pl.BlockSpec((BB, DIM), lambda i: (i, 0)),
            pl.BlockSpec((BB * NNB * DIM,), lambda i: (i,)),
            pl.BlockSpec((BB, NNB), lambda i: (i, 0)),
            pl.BlockSpec((DIM, DIM), lambda i: (0, 0)),
            pl.BlockSpec((1, DIM), lambda i: (0, 0)),
            pl.BlockSpec((DIM, DIM), lambda i: (0, 0)),
            pl.BlockSpec((1, DIM), lambda i: (0, 0)),
        ],
        out_specs=pl.BlockSpec((BB,), lambda i: (i,)),
        out_shape=jax.ShapeDtypeStruct((B,), jnp.float32),
    )(user_emb, sum0, sum1_2d, w0, W0T, b0, W1T, b1)


def kernel(u, v, adj, rel, train_mode, usr_table, ent_table, rel_table,
           agg_W0, agg_b0, agg_W1, agg_b1):
    del train_mode
    u = u.astype(jnp.int32)
    v = v.astype(jnp.int32)
    adj = adj.astype(jnp.int32)
    rel = rel.astype(jnp.int32)

    rtT = jnp.zeros((DIM, NRELP), jnp.float32).at[:, :rel_table.shape[0]].set(
        rel_table.T)

    user_emb, sum0, sum1, w0 = _gather_aggregate(
        u, v, adj, rel, usr_table, ent_table.astype(jnp.bfloat16), rtT)

    perm = jnp.concatenate([jnp.arange(0, DIM, 2), jnp.arange(1, DIM, 2)])
    return _dense_finish(
        user_emb, sum0, sum1.reshape(-1), w0,
        agg_W0.T[perm, :], agg_b0.reshape(1, DIM), agg_W1.T,
        agg_b1.reshape(1, DIM))
